# baseline (speedup 1.0000x reference)
import jax
import jax.numpy as jnp
from jax.experimental import pallas as pl
from jax.experimental.pallas import tpu as pltpu

N = 4096
F = 128
BI = 512
BJ = 512


def _make_hop_body(gi, gj, bi, bj, need_r, need_next, has_acc, relu):
    def body(*refs):
        it = iter(refs)
        A = next(it)
        x0 = next(it)
        x1 = next(it)
        accl = next(it) if has_acc else None
        accr = next(it) if (has_acc and need_r) else None
        W1 = next(it)
        b1 = next(it)
        W2 = next(it)
        b2 = next(it)
        outl = next(it)
        outr = next(it) if need_r else None
        nl = next(it) if need_next else None
        nr = next(it) if need_next else None
        yl = next(it)
        yr = next(it) if need_r else None

        i = pl.program_id(0)
        j = pl.program_id(1)
        a = A[...]

        x0b = x0[pl.ds(j * bj, bj), :]
        part_l = jax.lax.dot_general(
            a, x0b, (((1,), (0,)), ((), ())), preferred_element_type=jnp.float32
        )

        @pl.when(j == 0)
        def _():
            yl[pl.ds(i * bi, bi), :] = part_l

        @pl.when(j != 0)
        def _():
            yl[pl.ds(i * bi, bi), :] += part_l

        if need_r:
            x1b = x1[pl.ds(i * bi, bi), :]
            part_r = jax.lax.dot_general(
                a, x1b, (((0,), (0,)), ((), ())), preferred_element_type=jnp.float32
            )

            @pl.when(i == 0)
            def _():
                yr[pl.ds(j * bj, bj), :] = part_r

            @pl.when(i != 0)
            def _():
                yr[pl.ds(j * bj, bj), :] += part_r

        @pl.when((i == gi - 1) & (j == gj - 1))
        def _():
            W1v = W1[...]
            W2v = W2[...]
            bias = b1[...] + b2[...]
            ylv = yl[...]
            lm = ylv * x1[...]
            ol = (
                jnp.dot(ylv, W1v, preferred_element_type=jnp.float32)
                + jnp.dot(lm, W2v, preferred_element_type=jnp.float32)
                + bias
            )
            if has_acc:
                ol = ol + accl[...]
            if relu:
                ol = jnp.maximum(ol, 0.0)
            outl[...] = ol
            if need_next:
                nl[...] = ylv + lm
            if need_r:
                yrv = yr[...]
                rm = yrv * x0[...]
                orv = (
                    jnp.dot(yrv, W1v, preferred_element_type=jnp.float32)
                    + jnp.dot(rm, W2v, preferred_element_type=jnp.float32)
                    + bias
                )
                if has_acc:
                    orv = orv + accr[...]
                if relu:
                    orv = jnp.maximum(orv, 0.0)
                outr[...] = orv
                if need_next:
                    nr[...] = yrv + rm

    return body


def _hop(A, x0, x1, accs, W1, b1, W2, b2, *, need_r, need_next, relu):
    has_acc = accs is not None
    gi = N // BI
    gj = N // BJ
    full = pl.BlockSpec((N, F), lambda i, j: (0, 0))
    wspec = pl.BlockSpec((F, F), lambda i, j: (0, 0))
    bspec = pl.BlockSpec((1, F), lambda i, j: (0, 0))
    in_specs = [pl.BlockSpec((BI, BJ), lambda i, j: (i, j)), full, full]
    ops = [A, x0, x1]
    if has_acc:
        in_specs.append(full)
        ops.append(accs[0])
        if need_r:
            in_specs.append(full)
            ops.append(accs[1])
    in_specs += [wspec, bspec, wspec, bspec]
    ops += [W1, b1, W2, b2]

    n_outs = 1 + (1 if need_r else 0) + (2 if need_next else 0)
    out_shape = tuple(jax.ShapeDtypeStruct((N, F), jnp.float32) for _ in range(n_outs))
    out_specs = tuple(full for _ in range(n_outs))
    scratch = [pltpu.VMEM((N, F), jnp.float32)]
    if need_r:
        scratch.append(pltpu.VMEM((N, F), jnp.float32))

    return pl.pallas_call(
        _make_hop_body(gi, gj, BI, BJ, need_r, need_next, has_acc, relu),
        grid=(gi, gj),
        in_specs=in_specs,
        out_specs=out_specs,
        out_shape=out_shape,
        scratch_shapes=scratch,
    )(*ops)


def kernel(l_feat, r_feat, network, W1a, b1a, W2a, b2a, W1b, b1b, W2b, b2b):
    b1a = b1a.reshape(1, F)
    b2a = b2a.reshape(1, F)
    b1b = b1b.reshape(1, F)
    b2b = b2b.reshape(1, F)

    ol, orv, nl, nr = _hop(
        network, r_feat, l_feat, None, W1a, b1a, W2a, b2a,
        need_r=True, need_next=True, relu=False,
    )
    y1, z1 = _hop(
        network, nr, nl, (ol, orv), W1a, b1a, W2a, b2a,
        need_r=True, need_next=False, relu=True,
    )
    ol2, or2, nl2, nr2 = _hop(
        network, z1, y1, None, W1b, b1b, W2b, b2b,
        need_r=True, need_next=True, relu=False,
    )
    (y2,) = _hop(
        network, nr2, nl2, (ol2,), W1b, b1b, W2b, b2b,
        need_r=False, need_next=False, relu=True,
    )
    return y2
